# baseline (speedup 1.0000x reference)
import functools

import jax
import jax.numpy as jnp
import numpy as np
from jax import lax
from jax.experimental import pallas as pl
from jax.experimental.pallas import tpu as pltpu
from jax.experimental.pallas import tpu_sc as plsc

_INTERPRET = False
_F32 = jnp.float32


def _tap_wT(w_iohw):
    i, o, kh, kw = w_iohw.shape
    return jnp.transpose(w_iohw, (2, 3, 0, 1)).reshape(kh * kw, i, o)


def _gfold_w(w_cd, g):
    ci, co = w_cd.shape[2], w_cd.shape[3]
    blocks = []
    for dy in range(3):
        for s in range(3):
            b = jnp.zeros((g * ci, g * co), _F32)
            for dx in range(3):
                k = dx - 1 + (1 - s) * g
                if -g < k < g:
                    b = b + jnp.kron(jnp.eye(g, k=-k, dtype=_F32),
                                     w_cd[dy, dx])
            blocks.append(b)
    return jnp.stack(blocks)


def _gfold_s2_w(w_cd, g):
    g = int(g)
    e1 = np.zeros((2 * g, g), np.float32)
    e2 = np.zeros((2 * g, g), np.float32)
    e0 = np.zeros((2 * g, g), np.float32)
    el = np.zeros((2 * g, g), np.float32)
    for q in range(g):
        e1[2 * q, q] = 1
        e2[2 * q + 1, q] = 1
        if q + 1 < g:
            e0[2 * q + 1, q + 1] = 1
    el[2 * g - 1, 0] = 1
    mats = []
    for dy in range(3):
        mats.append(jnp.kron(jnp.asarray(el), w_cd[dy, 0]))
        mats.append(jnp.kron(jnp.asarray(e1), w_cd[dy, 1])
                    + jnp.kron(jnp.asarray(e2), w_cd[dy, 2])
                    + jnp.kron(jnp.asarray(e0), w_cd[dy, 0]))
    return jnp.stack(mats)


_CT_TAPS = {
    (0, 0): ((0, 0, 4),),
    (0, 1): ((0, 0, 5), (0, 1, 3)),
    (1, 0): ((0, 0, 7), (1, 0, 1)),
    (1, 1): ((0, 0, 8), (0, 1, 6), (1, 0, 2), (1, 1, 0)),
}
_CT_SPECS = ((0, 0, 0), (0, 0, 1), (1, 0, 0), (1, 0, 1), (1, 1, 0), (1, 1, 1))


def _gfoldt_w(w9, gi):
    gi = int(gi)

    def sel(sj, b):
        s = np.zeros((gi, 2 * gi), np.float32)
        for q in range(gi):
            if sj == 0:
                s[q, 2 * q + b] = 1
            elif q + 1 < gi:
                s[q + 1, 2 * q + b] = 1
        return s

    def selc(b):
        s = np.zeros((gi, 2 * gi), np.float32)
        s[0, 2 * (gi - 1) + b] = 1
        return s

    mats = []
    for a in (0, 1):
        for si in ((0,) if a == 0 else (0, 1)):
            c = jnp.zeros((gi * w9.shape[1], 2 * gi * w9.shape[2]), _F32)
            r = jnp.zeros((gi * w9.shape[1], 2 * gi * w9.shape[2]), _F32)
            for b in (0, 1):
                for (si2, sj, t) in _CT_TAPS[(a, b)]:
                    if si2 != si:
                        continue
                    if sj == 0:
                        c = c + jnp.kron(jnp.asarray(sel(0, b)), w9[t])
                    else:
                        c = c + jnp.kron(jnp.asarray(sel(1, b)), w9[t])
                        r = r + jnp.kron(jnp.asarray(selc(b)), w9[t])
            mats += [c, r]
    return jnp.stack(mats)


def _zero_borders(o_ref, uo):
    o_ref[0, 0] = jnp.zeros_like(o_ref[0, 0])
    for u in range(uo + 1, o_ref.shape[1]):
        o_ref[0, u] = jnp.zeros_like(o_ref[0, u])
    o_ref[0, :, :, 0, :] = jnp.zeros_like(o_ref[0, :, :, 0, :])
    o_ref[0, :, :, 29, :] = jnp.zeros_like(o_ref[0, :, :, 29, :])


def _conv1(xg, w9, b2):
    n = xg.shape[0]

    def body(x_ref, w_ref, b_ref, o_ref):
        acc = jnp.zeros((224 * 28, 128), _F32)
        for dy in range(3):
            for s in range(3):
                a = x_ref[0, dy:dy + 224, s:s + 28, :].reshape(224 * 28, 24)
                acc = acc + jnp.dot(a, w_ref[dy * 3 + s],
                                    preferred_element_type=_F32)
        v = jnp.maximum(acc + b_ref[0:1, :], 0.0)
        o_ref[0, 1:113, :, 1:29, :] = v.reshape(112, 2, 28, 128)
        _zero_borders(o_ref, 112)

    return pl.pallas_call(
        body,
        grid=(n,),
        in_specs=[pl.BlockSpec((1, 226, 30, 24), lambda i: (i, 0, 0, 0)),
                  pl.BlockSpec((9, 24, 128), lambda i: (0, 0, 0)),
                  pl.BlockSpec((1, 128), lambda i: (0, 0))],
        out_specs=pl.BlockSpec((1, 114, 2, 30, 128),
                               lambda i: (i, 0, 0, 0, 0)),
        out_shape=jax.ShapeDtypeStruct((n, 114, 2, 30, 128), _F32),
        interpret=_INTERPRET,
    )(xg, w9, b2)


def _conv_s2(x6, w6, b2, fold_out):
    n, u = x6.shape[0], x6.shape[1]
    ho, gco = u - 2, w6.shape[2]

    def body(x_ref, w_ref, b_ref, o_ref):
        acc = jnp.zeros((ho * 28, gco), _F32)
        for dy in range(3):
            di, pu = divmod(dy + 1, 2)
            for s in range(2):
                a = x_ref[0, di:di + ho, pu, s:s + 28, :].reshape(ho * 28, 128)
                acc = acc + jnp.dot(a, w_ref[dy * 2 + s],
                                    preferred_element_type=_F32)
        v = jnp.maximum(acc + b_ref[0:1, :], 0.0)
        if fold_out:
            o_ref[0, 1:1 + ho // 2, :, 1:29, :] = v.reshape(
                ho // 2, 2, 28, gco)
            _zero_borders(o_ref, ho // 2)
        else:
            o_ref[0] = v.reshape(ho, 28, gco)

    if fold_out:
        oshape = (n, ho // 2 + 2, 2, 30, gco)
        ospec = pl.BlockSpec((1,) + oshape[1:], lambda i: (i, 0, 0, 0, 0))
    else:
        oshape = (n, ho, 28, gco)
        ospec = pl.BlockSpec((1,) + oshape[1:], lambda i: (i, 0, 0, 0))
    return pl.pallas_call(
        body,
        grid=(n,),
        in_specs=[pl.BlockSpec((1, u, 2, 30, 128),
                               lambda i: (i, 0, 0, 0, 0)),
                  pl.BlockSpec((6, 128, gco), lambda i: (0, 0, 0)),
                  pl.BlockSpec((1, gco), lambda i: (0, 0))],
        out_specs=ospec,
        out_shape=jax.ShapeDtypeStruct(oshape, _F32),
        interpret=_INTERPRET,
    )(x6, w6, b2)


def _vq_idx(z, embed):
    n = z.shape[0]
    k, d = embed.shape

    def body(z_ref, e_ref, i_ref):
        zz = z_ref[0].reshape(784, d)
        e = e_ref[...]
        zd = jax.lax.dot_general(zz, e, (((1,), (1,)), ((), ())),
                                 preferred_element_type=_F32)
        e2 = jax.lax.dot_general(jnp.ones((1, d), _F32), e * e,
                                 (((1,), (1,)), ((), ())),
                                 preferred_element_type=_F32)
        z2 = jnp.sum(zz * zz, axis=1, keepdims=True)
        dist = z2 - 2.0 * zd + e2
        mn = jnp.min(dist, axis=1, keepdims=True)
        li = jax.lax.broadcasted_iota(jnp.int32, (784, k), 1)
        idx = jnp.min(jnp.where(dist == mn, li, k), axis=1, keepdims=True)
        i_ref[0, 0:784, :] = idx
        i_ref[0, 784:800, :] = jnp.zeros((16, 1), jnp.int32)

    return pl.pallas_call(
        body,
        grid=(n,),
        in_specs=[pl.BlockSpec((1, 28, 28, d), lambda i: (i, 0, 0, 0)),
                  pl.BlockSpec((k, d), lambda i: (0, 0))],
        out_specs=pl.BlockSpec((1, 800, 1), lambda i: (i, 0, 0)),
        out_shape=jax.ShapeDtypeStruct((n, 800, 1), jnp.int32),
        interpret=_INTERPRET,
    )(z, embed)


def _sc_gather(embed, idx_flat):
    b = idx_flat.shape[0]
    d = embed.shape[1]
    info = plsc.get_sparse_core_info()
    nc, ns = info.num_cores, info.num_subcores
    bpw = b // (nc * ns)
    mesh = plsc.VectorSubcoreMesh(core_axis_name="c", subcore_axis_name="s")

    @functools.partial(
        pl.kernel, mesh=mesh,
        out_type=jax.ShapeDtypeStruct((b, d), jnp.float32),
        scratch_types=[
            pltpu.VMEM((bpw,), jnp.int32),
            pltpu.VMEM((bpw, d), jnp.float32),
            pltpu.SemaphoreType.DMA,
        ],
    )
    def k(table_hbm, idx_hbm, out_hbm, idx_v, rows_v, sem):
        wid = lax.axis_index("s") * nc + lax.axis_index("c")
        base = wid * bpw
        pltpu.sync_copy(idx_hbm.at[pl.ds(base, bpw)], idx_v)
        pltpu.async_copy(table_hbm.at[idx_v], rows_v, sem).wait()
        pltpu.sync_copy(rows_v, out_hbm.at[pl.ds(base, bpw)])

    return k(embed, idx_flat)


def _convt1_sse(xg, w6, b2, z):
    n, hp1, wg1, gci = xg.shape
    hi, wg = hp1 - 1, wg1 - 1
    g2co = w6.shape[2]

    def body(x_ref, w_ref, b_ref, z_ref, o_ref, sse_ref):
        i = pl.program_id(0)
        for a in (0, 1):
            acc = jnp.zeros((hi * wg, g2co), _F32)
            for idx, (aa, si, gofs) in enumerate(_CT_SPECS):
                if aa != a:
                    continue
                v = x_ref[0, si:si + hi, gofs:gofs + wg, :].reshape(
                    hi * wg, gci)
                acc = acc + jnp.dot(v, w_ref[idx],
                                    preferred_element_type=_F32)
            r = jnp.maximum(acc + b_ref[0:1, :], 0.0)
            o_ref[0, :, a, :, :] = r.reshape(hi, wg, g2co)
        dlt = x_ref[0, 0:28, 0:28, :] - z_ref[0]

        @pl.when(i == 0)
        def _():
            sse_ref[0, 0] = 0.0

        sse_ref[0, 0] += jnp.sum(dlt * dlt)

    return pl.pallas_call(
        body,
        grid=(n,),
        in_specs=[pl.BlockSpec((1, hp1, wg1, gci), lambda i: (i, 0, 0, 0)),
                  pl.BlockSpec((6, gci, g2co), lambda i: (0, 0, 0)),
                  pl.BlockSpec((1, g2co), lambda i: (0, 0)),
                  pl.BlockSpec((1, 28, 28, gci), lambda i: (i, 0, 0, 0))],
        out_specs=[pl.BlockSpec((1, hi, 2, wg, g2co),
                                lambda i: (i, 0, 0, 0, 0)),
                   pl.BlockSpec((1, 1), lambda i: (0, 0),
                                memory_space=pltpu.SMEM)],
        out_shape=[jax.ShapeDtypeStruct((n, hi, 2, wg, g2co), _F32),
                   jax.ShapeDtypeStruct((1, 1), _F32)],
        interpret=_INTERPRET,
    )(xg, w6, b2, z)


def _convt_s2(xg, w6, b2):
    n, hp1, wg1, gci = xg.shape
    hi, wg = hp1 - 1, wg1 - 1
    g2co = w6.shape[2]

    def body(x_ref, w_ref, b_ref, o_ref):
        for a in (0, 1):
            acc = jnp.zeros((hi * wg, g2co), _F32)
            for idx, (aa, si, gofs) in enumerate(_CT_SPECS):
                if aa != a:
                    continue
                v = x_ref[0, si:si + hi, gofs:gofs + wg, :].reshape(
                    hi * wg, gci)
                acc = acc + jnp.dot(v, w_ref[idx],
                                    preferred_element_type=_F32)
            r = jnp.maximum(acc + b_ref[0:1, :], 0.0)
            o_ref[0, :, a, :, :] = r.reshape(hi, wg, g2co)

    return pl.pallas_call(
        body,
        grid=(n,),
        in_specs=[pl.BlockSpec((1, hp1, wg1, gci), lambda i: (i, 0, 0, 0)),
                  pl.BlockSpec((6, gci, g2co), lambda i: (0, 0, 0)),
                  pl.BlockSpec((1, g2co), lambda i: (0, 0))],
        out_specs=pl.BlockSpec((1, hi, 2, wg, g2co),
                               lambda i: (i, 0, 0, 0, 0)),
        out_shape=jax.ShapeDtypeStruct((n, hi, 2, wg, g2co), _F32),
        interpret=_INTERPRET,
    )(xg, w6, b2)


def _convt3(xg, w6, b2):
    n = xg.shape[0]
    hi, wg = 112, 28

    def body(x_ref, w_ref, b_ref, o_ref):
        for a in (0, 1):
            acc = jnp.zeros((hi * wg, 128), _F32)
            for idx, (aa, si, gofs) in enumerate(_CT_SPECS):
                if aa != a:
                    continue
                v = x_ref[0, si:si + hi, gofs:gofs + wg, :].reshape(
                    hi * wg, 128)
                acc = acc + jnp.dot(v, w_ref[idx],
                                    preferred_element_type=_F32)
            r = jnp.maximum(acc + b_ref[0:1, :], 0.0)
            o_ref[0, 1:113, a, 1:29, :] = r.reshape(hi, wg, 128)
        _zero_borders(o_ref, 112)

    return pl.pallas_call(
        body,
        grid=(n,),
        in_specs=[pl.BlockSpec((1, 113, 29, 128), lambda i: (i, 0, 0, 0)),
                  pl.BlockSpec((6, 128, 128), lambda i: (0, 0, 0)),
                  pl.BlockSpec((1, 128), lambda i: (0, 0))],
        out_specs=pl.BlockSpec((1, 115, 2, 30, 128),
                               lambda i: (i, 0, 0, 0, 0)),
        out_shape=jax.ShapeDtypeStruct((n, 115, 2, 30, 128), _F32),
        interpret=_INTERPRET,
    )(xg, w6, b2)


def _conv4t_loss(xf, w9, b2, target):
    n = xf.shape[0]

    def body(x_ref, w_ref, b_ref, t_ref, o_ref, sse_ref):
        i = pl.program_id(0)

        @pl.when(i == 0)
        def _():
            sse_ref[0, 0] = 0.0

        for c in (0, 1):
            acc = jnp.zeros((112 * 28, 24), _F32)
            for dy in range(3):
                q, pu = divmod(c + dy - 1, 2)
                u0 = 1 + q
                for s in range(3):
                    a = x_ref[0, u0:u0 + 112, pu, s:s + 28, :].reshape(
                        112 * 28, 128)
                    acc = acc + jnp.dot(a, w_ref[dy * 3 + s],
                                        preferred_element_type=_F32)
            v = acc + b_ref[0:1, :]
            o_ref[0, :, c, :, :] = v.reshape(112, 28, 24)
            dlt = v - t_ref[0, :, c, :, :].reshape(112 * 28, 24)
            sse_ref[0, 0] += jnp.sum(dlt * dlt)

    return pl.pallas_call(
        body,
        grid=(n,),
        in_specs=[pl.BlockSpec((1, 115, 2, 30, 128),
                               lambda i: (i, 0, 0, 0, 0)),
                  pl.BlockSpec((9, 128, 24), lambda i: (0, 0, 0)),
                  pl.BlockSpec((1, 24), lambda i: (0, 0)),
                  pl.BlockSpec((1, 112, 2, 28, 24),
                               lambda i: (i, 0, 0, 0, 0))],
        out_specs=[pl.BlockSpec((1, 112, 2, 28, 24),
                                lambda i: (i, 0, 0, 0, 0)),
                   pl.BlockSpec((1, 1), lambda i: (0, 0),
                                memory_space=pltpu.SMEM)],
        out_shape=[jax.ShapeDtypeStruct((n, 112, 2, 28, 24), _F32),
                   jax.ShapeDtypeStruct((1, 1), _F32)],
        interpret=_INTERPRET,
    )(xf, w9, b2, target)


def kernel(x, ew1, eb1, ew2, eb2, ew3, eb3, ew4, eb4,
           dw1, db1, dw2, db2, dw3, db3, dw4, db4, embed):
    n = x.shape[0]
    g = 8
    xh = jnp.transpose(x, (0, 2, 3, 1))

    xg = jnp.pad(xh, ((0, 0), (1, 1), (g, g), (0, 0))).reshape(n, 226, 30, 24)
    w1 = _gfold_w(jnp.transpose(ew1, (2, 3, 1, 0)), g)
    a2 = _conv1(xg, w1, jnp.tile(eb1, g)[None, :])
    w2 = _gfold_s2_w(jnp.transpose(ew2, (2, 3, 1, 0)), 4)
    a3 = _conv_s2(a2, w2, jnp.tile(eb2, 4)[None, :], True)
    w3 = _gfold_s2_w(jnp.transpose(ew3, (2, 3, 1, 0)), 2)
    a4 = _conv_s2(a3, w3, jnp.tile(eb3, 2)[None, :], True)
    w4e = _gfold_s2_w(jnp.transpose(ew4, (2, 3, 1, 0)), 1)
    z = _conv_s2(a4, w4e, eb4[None, :], False)

    idx = _vq_idx(z, embed)
    zq_flat = _sc_gather(embed, idx.reshape(-1))
    zq = zq_flat.reshape(n, 800, 128)[:, 0:784, :].reshape(n, 28, 28, 128)
    zqp = jnp.pad(zq, ((0, 0), (0, 1), (0, 1), (0, 0)))

    wt1 = _gfoldt_w(_tap_wT(dw1), 1)
    d1, sse_vq = _convt1_sse(zqp, wt1, jnp.tile(db1, 2)[None, :], z)
    diff = 2.0 * sse_vq[0, 0] / float(z.size)
    d1 = d1.reshape(n, 56, 56, 64)
    t2in = jnp.pad(d1, ((0, 0), (0, 1), (0, 2), (0, 0))).reshape(
        n, 57, 29, 128)
    wt2 = _gfoldt_w(_tap_wT(dw2), 2)
    d2 = _convt_s2(t2in, wt2, jnp.tile(db2, 4)[None, :])
    d2 = d2.reshape(n, 112, 112, 32)
    t3in = jnp.pad(d2, ((0, 0), (0, 1), (0, 4), (0, 0))).reshape(
        n, 113, 29, 128)
    wt3 = _gfoldt_w(_tap_wT(dw3), 4)
    d3f = _convt3(t3in, wt3, jnp.tile(db3, 8)[None, :])

    w4 = _gfold_w(jnp.transpose(jnp.flip(dw4, (2, 3)), (2, 3, 0, 1)), g)
    xh_f = xh.reshape(n, 112, 2, 28, 24)
    x_rec_f, sse_rec = _conv4t_loss(d3f, w4, jnp.tile(db4, g)[None, :], xh_f)

    x_rec = jnp.transpose(x_rec_f.reshape(n, 224, 224, 3), (0, 3, 1, 2))
    loss = sse_rec[0, 0] / float(x_rec.size) + 0.25 * diff
    return (x_rec, loss)
